# baseline (speedup 1.0000x reference)
import jax
import jax.numpy as jnp
from jax import lax
from jax.experimental import pallas as pl
from jax.experimental.pallas import tpu as pltpu

N = 10000
E = 320000
F = 128
D = 9

G_ROWS = 90112
EB = 8000
NSTEP = E // EB


def _nf_stage(feature, norm):
    def body(f_ref, n_ref, o_ref):
        o_ref[...] = f_ref[...] * n_ref[...]

    return pl.pallas_call(
        body,
        grid=(10,),
        in_specs=[
            pl.BlockSpec((N // 10, F), lambda i: (i, 0)),
            pl.BlockSpec((N // 10, 1), lambda i: (i, 0)),
        ],
        out_specs=pl.BlockSpec((N // 10, F), lambda i: (i, 0)),
        out_shape=jax.ShapeDtypeStruct((N, F), jnp.float32),
    )(feature, norm)


def _scatter_stage(nf, edges):
    def body(e_ref, nf_ref, o_ref):
        @pl.when(pl.program_id(0) == 0)
        def _init():
            o_ref[...] = jnp.zeros_like(o_ref)

        def edge_it(i, _):
            sv = e_ref[0, 0, i]
            r = e_ref[0, 1, i]
            msg = nf_ref[pl.ds(sv, 1), :]
            o_ref[pl.ds(r, 1), :] += msg
            return 0

        lax.fori_loop(0, EB, edge_it, 0, unroll=32)

    return pl.pallas_call(
        body,
        grid=(NSTEP,),
        in_specs=[
            pl.BlockSpec((1, 2, EB), lambda i: (i, 0, 0),
                         memory_space=pltpu.SMEM),
            pl.BlockSpec((N, F), lambda i: (0, 0)),
        ],
        out_specs=pl.BlockSpec((G_ROWS, F), lambda i: (0, 0)),
        out_shape=jax.ShapeDtypeStruct((G_ROWS, F), jnp.float32),
        compiler_params=pltpu.CompilerParams(
            vmem_limit_bytes=100 * 1024 * 1024),
    )(edges, nf)


def _matmul_stage(g3, w, norm):
    rb = N // 10

    def body(g_ref, w_ref, n_ref, o_ref):
        acc = lax.dot_general(
            g_ref[0], w_ref[0], (((1,), (1,)), ((), ())),
            preferred_element_type=jnp.float32)
        o_ref[...] = jnp.maximum(acc * n_ref[...], 0.0)

    return pl.pallas_call(
        body,
        grid=(D, 10),
        in_specs=[
            pl.BlockSpec((1, rb, F), lambda d, i: (d, i, 0)),
            pl.BlockSpec((1, F, F), lambda d, i: (d, 0, 0)),
            pl.BlockSpec((rb, 1), lambda d, i: (i, 0)),
        ],
        out_specs=pl.BlockSpec((rb, F), lambda d, i: (i, d)),
        out_shape=jax.ShapeDtypeStruct((N, D * F), jnp.float32),
    )(g3, w, norm)


def kernel(feature, edge_index, subgraph_idx, norm, W):
    nf = _nf_stage(feature, norm)
    rows = subgraph_idx * N + edge_index[1]
    edges = (jnp.stack([edge_index[0], rows])
             .reshape(2, NSTEP, EB).transpose(1, 0, 2))
    gflat = _scatter_stage(nf, edges)
    g3 = gflat[: D * N].reshape(D, N, F)
    return _matmul_stage(g3, W, norm)


# speedup vs baseline: 1.0036x; 1.0036x over previous
import jax
import jax.numpy as jnp
from jax import lax
from jax.experimental import pallas as pl
from jax.experimental.pallas import tpu as pltpu

N = 10000
E = 320000
F = 128
D = 9

G_ROWS = 90112
EB = 8000
NSTEP = E // EB


def _nf_stage(feature, norm):
    def body(f_ref, n_ref, o_ref):
        o_ref[...] = f_ref[...] * n_ref[...]

    return pl.pallas_call(
        body,
        grid=(10,),
        in_specs=[
            pl.BlockSpec((N // 10, F), lambda i: (i, 0)),
            pl.BlockSpec((N // 10, 1), lambda i: (i, 0)),
        ],
        out_specs=pl.BlockSpec((N // 10, F), lambda i: (i, 0)),
        out_shape=jax.ShapeDtypeStruct((N, F), jnp.float32),
    )(feature, norm)


def _scatter_stage(nf, edges):
    def body(e_ref, nf_ref, o_ref):
        @pl.when(pl.program_id(0) == 0)
        def _init():
            o_ref[...] = jnp.zeros_like(o_ref)

        def edge_it(i, _):
            sv = e_ref[0, 0, i]
            r = e_ref[0, 1, i]
            msg = nf_ref[pl.ds(sv, 1), :]
            o_ref[pl.ds(r, 1), :] += msg
            return 0

        lax.fori_loop(0, EB, edge_it, 0, unroll=16)

    return pl.pallas_call(
        body,
        grid=(NSTEP,),
        in_specs=[
            pl.BlockSpec((1, 2, EB), lambda i: (i, 0, 0),
                         memory_space=pltpu.SMEM),
            pl.BlockSpec((N, F), lambda i: (0, 0)),
        ],
        out_specs=pl.BlockSpec((G_ROWS, F), lambda i: (0, 0)),
        out_shape=jax.ShapeDtypeStruct((G_ROWS, F), jnp.float32),
        compiler_params=pltpu.CompilerParams(
            vmem_limit_bytes=100 * 1024 * 1024),
    )(edges, nf)


def _matmul_stage(g3, w, norm):
    rb = N // 10

    def body(g_ref, w_ref, n_ref, o_ref):
        acc = lax.dot_general(
            g_ref[0], w_ref[0], (((1,), (1,)), ((), ())),
            preferred_element_type=jnp.float32)
        o_ref[...] = jnp.maximum(acc * n_ref[...], 0.0)

    return pl.pallas_call(
        body,
        grid=(D, 10),
        in_specs=[
            pl.BlockSpec((1, rb, F), lambda d, i: (d, i, 0)),
            pl.BlockSpec((1, F, F), lambda d, i: (d, 0, 0)),
            pl.BlockSpec((rb, 1), lambda d, i: (i, 0)),
        ],
        out_specs=pl.BlockSpec((rb, F), lambda d, i: (i, d)),
        out_shape=jax.ShapeDtypeStruct((N, D * F), jnp.float32),
    )(g3, w, norm)


def kernel(feature, edge_index, subgraph_idx, norm, W):
    nf = _nf_stage(feature, norm)
    rows = subgraph_idx * N + edge_index[1]
    edges = (jnp.stack([edge_index[0], rows])
             .reshape(2, NSTEP, EB).transpose(1, 0, 2))
    gflat = _scatter_stage(nf, edges)
    g3 = gflat[: D * N].reshape(D, N, F)
    return _matmul_stage(g3, W, norm)
